# baseline (speedup 1.0000x reference)
import functools

import jax
import jax.numpy as jnp
from jax import lax
from jax.experimental import pallas as pl
from jax.experimental.pallas import tpu as pltpu
from jax.experimental.pallas import tpu_sc as plsc

NUM_PARENTS = 8
CHILDREN_PER_PARENT = 16
NUM_CLASSES = NUM_PARENTS + NUM_PARENTS * CHILDREN_PER_PARENT
BATCH = 16384

NC = 2
NS = 16
L = 16
NW = NC * NS
ROWS_PER_TILE = BATCH // NW
RBLK = 128
NBLK = ROWS_PER_TILE // RBLK
NCHUNK = RBLK // L

MAIN = 128
NTAIL = NUM_CLASSES - MAIN
TROWS = BATCH * NTAIL // 128

TCR = 256

_LN2 = 0.6931471805599453


def _vlog(x):
    ib = lax.bitcast_convert_type(x, jnp.int32)
    ex = lax.shift_right_logical(ib, 23) - 127
    mb = (ib & 0x007FFFFF) | 0x3F800000
    m = lax.bitcast_convert_type(mb, jnp.float32)
    big = m > 1.4142135
    m = jnp.where(big, m * 0.5, m)
    ef = ex.astype(jnp.float32) + jnp.where(big, 1.0, 0.0)
    z = (m - 1.0) / (m + 1.0)
    z2 = z * z
    p = 2.0 + z2 * (0.66666667 + z2 * (0.4 + z2 * (0.28571429 + z2 * 0.22222222)))
    return ef * _LN2 + z * p


def _tree_reduce(op, vs):
    vs = list(vs)
    while len(vs) > 1:
        nxt = [op(vs[i], vs[i + 1]) for i in range(0, len(vs) - 1, 2)]
        if len(vs) % 2:
            nxt.append(vs[-1])
        vs = nxt
    return vs[0]


def _split_body(p_ref, t_ref, pm_ref, pt_ref, tm_ref, tt_ref):
    pm_ref[...] = p_ref[:, :MAIN]
    pt_ref[...] = p_ref[:, MAIN:].T
    tm_ref[...] = t_ref[:, :MAIN]
    tt_ref[...] = t_ref[:, MAIN:].T


def _join_body(om_ref, ot_ref, o_ref):
    o_ref[:, :MAIN] = om_ref[...]
    o_ref[:, MAIN:] = ot_ref[...].T


@functools.cache
def _build_tc_split():
    g = BATCH // TCR
    tr = TCR * NTAIL // 128
    return pl.pallas_call(
        _split_body,
        grid=(g,),
        in_specs=[pl.BlockSpec((TCR, NUM_CLASSES), lambda i: (i, 0)),
                  pl.BlockSpec((TCR, NUM_CLASSES), lambda i: (i, 0))],
        out_specs=[pl.BlockSpec((TCR, MAIN), lambda i: (i, 0)),
                   pl.BlockSpec((NTAIL, TCR), lambda i: (0, i)),
                   pl.BlockSpec((TCR, MAIN), lambda i: (i, 0)),
                   pl.BlockSpec((NTAIL, TCR), lambda i: (0, i))],
        out_shape=[jax.ShapeDtypeStruct((BATCH, MAIN), jnp.float32),
                   jax.ShapeDtypeStruct((NTAIL, BATCH), jnp.float32),
                   jax.ShapeDtypeStruct((BATCH, MAIN), jnp.float32),
                   jax.ShapeDtypeStruct((NTAIL, BATCH), jnp.float32)],
    )


@functools.cache
def _build_tc_join():
    g = BATCH // TCR
    tr = TCR * NTAIL // 128
    return pl.pallas_call(
        _join_body,
        grid=(g,),
        in_specs=[pl.BlockSpec((TCR, MAIN), lambda i: (i, 0)),
                  pl.BlockSpec((NTAIL, TCR), lambda i: (0, i))],
        out_specs=pl.BlockSpec((TCR, NUM_CLASSES), lambda i: (i, 0)),
        out_shape=jax.ShapeDtypeStruct((BATCH, NUM_CLASSES), jnp.float32),
    )


@functools.cache
def _build_sc_kernel():
    return pl.kernel(
        _sc_hier_softmax,
        out_type=[
            jax.ShapeDtypeStruct((BATCH * MAIN,), jnp.float32),
            jax.ShapeDtypeStruct((NTAIL, BATCH), jnp.float32),
            jax.ShapeDtypeStruct((NW, L), jnp.float32),
        ],
        mesh=plsc.VectorSubcoreMesh(core_axis_name="c", subcore_axis_name="s",
                                    num_cores=NC, num_subcores=NS),
        compiler_params=pltpu.CompilerParams(needs_layout_passes=False),
        scratch_types=[
            pltpu.VMEM((RBLK * MAIN,), jnp.float32),
            pltpu.VMEM((RBLK * MAIN,), jnp.float32),
            pltpu.VMEM((RBLK * MAIN,), jnp.float32),
            pltpu.VMEM((NTAIL, RBLK), jnp.float32),
            pltpu.VMEM((NTAIL, RBLK), jnp.float32),
            pltpu.VMEM((NTAIL, RBLK), jnp.float32),
            pltpu.VMEM((NUM_PARENTS * L,), jnp.float32),
            pltpu.VMEM((L,), jnp.float32),
        ],
    )


def _sc_hier_softmax(pm_hbm, pt_hbm, tm_hbm, tt_hbm, om_hbm, ot_hbm, part_hbm,
                     pm_vm, tm_vm, om_vm, pt_vm, tt_vm, ot_vm, plp_vm, acc_vm):
    wid = lax.axis_index("s") * NC + lax.axis_index("c")
    base_row = wid * ROWS_PER_TILE
    lanes = lax.iota(jnp.int32, L)

    def chunk_body(ch, acc):
        rowbase = (ch * L + lanes) * MAIN

        idxs = [rowbase + c for c in range(NUM_PARENTS)]
        pv = [plsc.load_gather(pm_vm, [idx]) for idx in idxs]
        tv = [plsc.load_gather(tm_vm, [idx]) for idx in idxs]
        m = _tree_reduce(jnp.maximum, pv)
        s_vt = _tree_reduce(jnp.add, [v * t for v, t in zip(pv, tv)])
        s_t = _tree_reduce(jnp.add, tv)
        ev = [jnp.exp(v - m) for v in pv]
        s = _tree_reduce(jnp.add, ev)
        log_z = m + _vlog(s)
        plp = [v - log_z for v in pv]
        acc = acc + (s_vt - log_z * s_t)
        rcp = 1.0 / s
        for c in range(NUM_PARENTS):
            plsc.store_scatter(om_vm, [idxs[c]], ev[c] * rcp)

        for c in range(NUM_PARENTS):
            plp_vm[pl.ds(c * L, L)] = plp[c]

        def group_body(g, acc):
            col0 = NUM_PARENTS + g * CHILDREN_PER_PARENT
            cb = rowbase + col0
            idxs = [cb + c for c in range(CHILDREN_PER_PARENT)]
            plp_g = plp_vm[pl.ds(g * L, L)]
            vs = [plsc.load_gather(pm_vm, [idx]) for idx in idxs]
            ts = [plsc.load_gather(tm_vm, [idx]) for idx in idxs]
            mg = _tree_reduce(jnp.maximum, vs)
            s_vt = _tree_reduce(jnp.add, [v * t for v, t in zip(vs, ts)])
            s_t = _tree_reduce(jnp.add, ts)
            eg = [jnp.exp(v - mg) for v in vs]
            sg = _tree_reduce(jnp.add, eg)
            base = mg + _vlog(sg) - plp_g
            acc = acc + (s_vt - base * s_t)
            pf = jnp.exp(plp_g) / sg
            for c in range(CHILDREN_PER_PARENT):
                plsc.store_scatter(om_vm, [idxs[c]], eg[c] * pf)
            return acc

        acc = lax.fori_loop(0, NUM_PARENTS - 1, group_body, acc)

        idxm = [rowbase + (MAIN - NTAIL) + c for c in range(NTAIL)]
        tcol = ch * L + lanes
        jrows = [jnp.full((L,), j, jnp.int32) for j in range(NTAIL)]
        vs = ([plsc.load_gather(pm_vm, [idx]) for idx in idxm]
              + [plsc.load_gather(pt_vm, [jr, tcol]) for jr in jrows])
        ts = ([plsc.load_gather(tm_vm, [idx]) for idx in idxm]
              + [plsc.load_gather(tt_vm, [jr, tcol]) for jr in jrows])
        mg = _tree_reduce(jnp.maximum, vs)
        s_vt = _tree_reduce(jnp.add, [v * t for v, t in zip(vs, ts)])
        s_t = _tree_reduce(jnp.add, ts)
        eg = [jnp.exp(v - mg) for v in vs]
        sg = _tree_reduce(jnp.add, eg)
        base = mg + _vlog(sg) - plp[NUM_PARENTS - 1]
        acc = acc + (s_vt - base * s_t)
        pf = jnp.exp(plp[NUM_PARENTS - 1]) / sg
        for c in range(NTAIL):
            plsc.store_scatter(om_vm, [idxm[c]], eg[c] * pf)
        for j in range(NTAIL):
            plsc.store_scatter(ot_vm, [jrows[j], tcol], eg[NTAIL + j] * pf)
        return acc

    def block_body(blk, acc):
        rowoff = base_row + blk * RBLK
        off_m = rowoff * MAIN
        pltpu.sync_copy(pm_hbm.at[pl.ds(off_m, RBLK * MAIN)], pm_vm)
        pltpu.sync_copy(tm_hbm.at[pl.ds(off_m, RBLK * MAIN)], tm_vm)
        pltpu.sync_copy(pt_hbm.at[:, pl.ds(rowoff, RBLK)], pt_vm)
        pltpu.sync_copy(tt_hbm.at[:, pl.ds(rowoff, RBLK)], tt_vm)
        acc = lax.fori_loop(0, NCHUNK, chunk_body, acc)
        pltpu.sync_copy(om_vm, om_hbm.at[pl.ds(off_m, RBLK * MAIN)])
        pltpu.sync_copy(ot_vm, ot_hbm.at[:, pl.ds(rowoff, RBLK)])
        return acc

    acc = lax.fori_loop(0, NBLK, block_body, jnp.zeros((L,), jnp.float32))
    acc_vm[...] = acc
    pltpu.sync_copy(acc_vm, part_hbm.at[wid])


def kernel(pred, target, _):
    pm, pt, tm, tt = _build_tc_split()(pred, target)
    om_f, ot_f, parts = _build_sc_kernel()(
        pm.reshape(-1), pt, tm.reshape(-1), tt)
    out = _build_tc_join()(om_f.reshape(BATCH, MAIN), ot_f)
    loss = -(parts.sum() / BATCH)
    return (loss, out)


# speedup vs baseline: 1.7625x; 1.7625x over previous
import functools

import jax
import jax.numpy as jnp
from jax import lax
from jax.experimental import pallas as pl
from jax.experimental.pallas import tpu as pltpu
from jax.experimental.pallas import tpu_sc as plsc

NUM_PARENTS = 8
CHILDREN_PER_PARENT = 16
NUM_CLASSES = NUM_PARENTS + NUM_PARENTS * CHILDREN_PER_PARENT
BATCH = 16384

NC = 2
NS = 16
L = 16
NW = NC * NS
ROWS_PER_TILE = BATCH // NW
RBLK = 128
NBLK = ROWS_PER_TILE // RBLK
NCHUNK = RBLK // L

_LN2 = 0.6931471805599453


def _vlog(x):
    ib = lax.bitcast_convert_type(x, jnp.int32)
    ex = lax.shift_right_logical(ib, 23) - 127
    mb = (ib & 0x007FFFFF) | 0x3F800000
    m = lax.bitcast_convert_type(mb, jnp.float32)
    big = m > 1.4142135
    m = jnp.where(big, m * 0.5, m)
    ef = ex.astype(jnp.float32) + jnp.where(big, 1.0, 0.0)
    z = (m - 1.0) / (m + 1.0)
    z2 = z * z
    p = 2.0 + z2 * (0.66666667 + z2 * (0.4 + z2 * (0.28571429 + z2 * 0.22222222)))
    return ef * _LN2 + z * p


def _tree_reduce(op, vs):
    vs = list(vs)
    while len(vs) > 1:
        nxt = [op(vs[i], vs[i + 1]) for i in range(0, len(vs) - 1, 2)]
        if len(vs) % 2:
            nxt.append(vs[-1])
        vs = nxt
    return vs[0]


@functools.cache
def _build_sc_kernel():
    return pl.kernel(
        _sc_hier_softmax,
        out_type=[
            jax.ShapeDtypeStruct((BATCH * NUM_CLASSES,), jnp.float32),
            jax.ShapeDtypeStruct((NW, L), jnp.float32),
        ],
        mesh=plsc.VectorSubcoreMesh(core_axis_name="c", subcore_axis_name="s",
                                    num_cores=NC, num_subcores=NS),
        compiler_params=pltpu.CompilerParams(needs_layout_passes=False),
        scratch_types=[
            pltpu.VMEM((RBLK * NUM_CLASSES,), jnp.float32),
            pltpu.VMEM((RBLK * NUM_CLASSES,), jnp.float32),
            pltpu.VMEM((RBLK * NUM_CLASSES,), jnp.float32),
            pltpu.VMEM((RBLK * NUM_CLASSES,), jnp.float32),
            pltpu.VMEM((RBLK * NUM_CLASSES,), jnp.float32),
            pltpu.VMEM((NUM_PARENTS * L,), jnp.float32),
            pltpu.VMEM((L,), jnp.float32),
            pltpu.SemaphoreType.DMA,
            pltpu.SemaphoreType.DMA,
        ],
    )


def _sc_hier_softmax(pred_hbm, targ_hbm, out_hbm, part_hbm,
                     pred_vm0, targ_vm0, pred_vm1, targ_vm1, out_vm,
                     plp_vm, acc_vm, sem0, sem1):
    wid = lax.axis_index("s") * NC + lax.axis_index("c")
    base_row = wid * ROWS_PER_TILE
    lanes = lax.iota(jnp.int32, L)
    nel = RBLK * NUM_CLASSES

    def chunk(pred_vm, targ_vm):
        def chunk_body(ch, acc):
            rowbase = (ch * L + lanes) * NUM_CLASSES

            idxs = [rowbase + c for c in range(NUM_PARENTS)]
            pv = [plsc.load_gather(pred_vm, [idx]) for idx in idxs]
            tv = [plsc.load_gather(targ_vm, [idx]) for idx in idxs]
            m = _tree_reduce(jnp.maximum, pv)
            s_vt = _tree_reduce(jnp.add, [v * t for v, t in zip(pv, tv)])
            s_t = _tree_reduce(jnp.add, tv)
            ev = [jnp.exp(v - m) for v in pv]
            s = _tree_reduce(jnp.add, ev)
            log_z = m + _vlog(s)
            plp = [v - log_z for v in pv]
            acc = acc + (s_vt - log_z * s_t)
            rcp = 1.0 / s
            for c in range(NUM_PARENTS):
                plsc.store_scatter(out_vm, [idxs[c]], ev[c] * rcp)

            for c in range(NUM_PARENTS):
                plp_vm[pl.ds(c * L, L)] = plp[c]

            def group_body(g, acc):
                col0 = NUM_PARENTS + g * CHILDREN_PER_PARENT
                cb = rowbase + col0
                idxs = [cb + c for c in range(CHILDREN_PER_PARENT)]
                plp_g = plp_vm[pl.ds(g * L, L)]
                vs = [plsc.load_gather(pred_vm, [idx]) for idx in idxs]
                ts = [plsc.load_gather(targ_vm, [idx]) for idx in idxs]
                mg = _tree_reduce(jnp.maximum, vs)
                s_vt = _tree_reduce(jnp.add, [v * t for v, t in zip(vs, ts)])
                s_t = _tree_reduce(jnp.add, ts)
                eg = [jnp.exp(v - mg) for v in vs]
                sg = _tree_reduce(jnp.add, eg)
                base = mg + _vlog(sg) - plp_g
                acc = acc + (s_vt - base * s_t)
                pf = jnp.exp(plp_g) / sg
                for c in range(CHILDREN_PER_PARENT):
                    plsc.store_scatter(out_vm, [idxs[c]], eg[c] * pf)
                return acc

            return lax.fori_loop(0, NUM_PARENTS, group_body, acc)
        return chunk_body

    def start_in(blk, pred_vm, targ_vm, sem):
        off = (base_row + blk * RBLK) * NUM_CLASSES
        pltpu.async_copy(pred_hbm.at[pl.ds(off, nel)], pred_vm, sem)
        pltpu.async_copy(targ_hbm.at[pl.ds(off, nel)], targ_vm, sem)

    def wait_in(pred_vm, targ_vm, sem):
        pltpu.make_async_copy(pred_hbm.at[pl.ds(0, nel)], pred_vm, sem).wait()
        pltpu.make_async_copy(targ_hbm.at[pl.ds(0, nel)], targ_vm, sem).wait()

    bufs = ((pred_vm0, targ_vm0, sem0), (pred_vm1, targ_vm1, sem1))
    start_in(0, *bufs[0])
    acc = jnp.zeros((L,), jnp.float32)
    for blk in range(NBLK):
        if blk + 1 < NBLK:
            start_in(blk + 1, *bufs[(blk + 1) % 2])
        pred_vm, targ_vm, sem = bufs[blk % 2]
        wait_in(pred_vm, targ_vm, sem)
        acc = lax.fori_loop(0, NCHUNK, chunk(pred_vm, targ_vm), acc)
        off = (base_row + blk * RBLK) * NUM_CLASSES
        pltpu.sync_copy(out_vm, out_hbm.at[pl.ds(off, nel)])
    acc_vm[...] = acc
    pltpu.sync_copy(acc_vm, part_hbm.at[wid])


def kernel(pred, target, _):
    out_flat, parts = _build_sc_kernel()(pred.reshape(-1), target.reshape(-1))
    loss = -(parts.sum() / BATCH)
    return (loss, out_flat.reshape(BATCH, NUM_CLASSES))


# speedup vs baseline: 1.7883x; 1.0146x over previous
import functools

import jax
import jax.numpy as jnp
from jax import lax
from jax.experimental import pallas as pl
from jax.experimental.pallas import tpu as pltpu
from jax.experimental.pallas import tpu_sc as plsc

NUM_PARENTS = 8
CHILDREN_PER_PARENT = 16
NUM_CLASSES = NUM_PARENTS + NUM_PARENTS * CHILDREN_PER_PARENT
BATCH = 16384

NC = 2
NS = 16
L = 16
NW = NC * NS
ROWS_PER_TILE = BATCH // NW
RBLK = 128
NBLK = ROWS_PER_TILE // RBLK
NCHUNK = RBLK // L

_LN2 = 0.6931471805599453


def _vlog(x):
    ib = lax.bitcast_convert_type(x, jnp.int32)
    ex = lax.shift_right_logical(ib, 23) - 127
    mb = (ib & 0x007FFFFF) | 0x3F800000
    m = lax.bitcast_convert_type(mb, jnp.float32)
    big = m > 1.4142135
    m = jnp.where(big, m * 0.5, m)
    ef = ex.astype(jnp.float32) + jnp.where(big, 1.0, 0.0)
    z = (m - 1.0) / (m + 1.0)
    z2 = z * z
    p = 2.0 + z2 * (0.66666667 + z2 * (0.4 + z2 * (0.28571429 + z2 * 0.22222222)))
    return ef * _LN2 + z * p


def _tree_reduce(op, vs):
    vs = list(vs)
    while len(vs) > 1:
        nxt = [op(vs[i], vs[i + 1]) for i in range(0, len(vs) - 1, 2)]
        if len(vs) % 2:
            nxt.append(vs[-1])
        vs = nxt
    return vs[0]


@functools.cache
def _build_sc_kernel():
    return pl.kernel(
        _sc_hier_softmax,
        out_type=[
            jax.ShapeDtypeStruct((BATCH * NUM_CLASSES,), jnp.float32),
            jax.ShapeDtypeStruct((NW, L), jnp.float32),
        ],
        mesh=plsc.VectorSubcoreMesh(core_axis_name="c", subcore_axis_name="s",
                                    num_cores=NC, num_subcores=NS),
        compiler_params=pltpu.CompilerParams(needs_layout_passes=False),
        scratch_types=[
            pltpu.VMEM((RBLK * NUM_CLASSES,), jnp.float32),
            pltpu.VMEM((RBLK * NUM_CLASSES,), jnp.float32),
            pltpu.VMEM((RBLK * NUM_CLASSES,), jnp.float32),
            pltpu.VMEM((RBLK * NUM_CLASSES,), jnp.float32),
            pltpu.VMEM((RBLK * NUM_CLASSES,), jnp.float32),
            pltpu.VMEM((RBLK * NUM_CLASSES,), jnp.float32),
            pltpu.VMEM((NUM_PARENTS * L,), jnp.float32),
            pltpu.VMEM((L,), jnp.float32),
            pltpu.SemaphoreType.DMA,
            pltpu.SemaphoreType.DMA,
            pltpu.SemaphoreType.DMA,
            pltpu.SemaphoreType.DMA,
        ],
    )


def _sc_hier_softmax(pred_hbm, targ_hbm, out_hbm, part_hbm,
                     pred_vm0, targ_vm0, pred_vm1, targ_vm1, out_vm0, out_vm1,
                     plp_vm, acc_vm, sem0, sem1, semo0, semo1):
    wid = lax.axis_index("s") * NC + lax.axis_index("c")
    base_row = wid * ROWS_PER_TILE
    lanes = lax.iota(jnp.int32, L)
    nel = RBLK * NUM_CLASSES

    def chunk(pred_vm, targ_vm, out_vm):
        def chunk_body(ch, acc):
            rowbase = (ch * L + lanes) * NUM_CLASSES

            idxs = [rowbase + c for c in range(NUM_PARENTS)]
            pv = [plsc.load_gather(pred_vm, [idx]) for idx in idxs]
            tv = [plsc.load_gather(targ_vm, [idx]) for idx in idxs]
            m = _tree_reduce(jnp.maximum, pv)
            s_vt = _tree_reduce(jnp.add, [v * t for v, t in zip(pv, tv)])
            s_t = _tree_reduce(jnp.add, tv)
            ev = [jnp.exp(v - m) for v in pv]
            s = _tree_reduce(jnp.add, ev)
            log_z = m + _vlog(s)
            plp = [v - log_z for v in pv]
            acc = acc + (s_vt - log_z * s_t)
            rcp = 1.0 / s
            for c in range(NUM_PARENTS):
                plsc.store_scatter(out_vm, [idxs[c]], ev[c] * rcp)

            for c in range(NUM_PARENTS):
                plp_vm[pl.ds(c * L, L)] = plp[c]

            def group_body(g, acc):
                col0 = NUM_PARENTS + g * CHILDREN_PER_PARENT
                cb = rowbase + col0
                idxs = [cb + c for c in range(CHILDREN_PER_PARENT)]
                plp_g = plp_vm[pl.ds(g * L, L)]
                vs = [plsc.load_gather(pred_vm, [idx]) for idx in idxs]
                ts = [plsc.load_gather(targ_vm, [idx]) for idx in idxs]
                mg = _tree_reduce(jnp.maximum, vs)
                s_vt = _tree_reduce(jnp.add, [v * t for v, t in zip(vs, ts)])
                s_t = _tree_reduce(jnp.add, ts)
                eg = [jnp.exp(v - mg) for v in vs]
                sg = _tree_reduce(jnp.add, eg)
                base = mg + _vlog(sg) - plp_g
                acc = acc + (s_vt - base * s_t)
                pf = jnp.exp(plp_g) / sg
                for c in range(CHILDREN_PER_PARENT):
                    plsc.store_scatter(out_vm, [idxs[c]], eg[c] * pf)
                return acc

            return lax.fori_loop(0, NUM_PARENTS, group_body, acc)
        return chunk_body

    def start_in(blk, pred_vm, targ_vm, sem):
        off = (base_row + blk * RBLK) * NUM_CLASSES
        pltpu.async_copy(pred_hbm.at[pl.ds(off, nel)], pred_vm, sem)
        pltpu.async_copy(targ_hbm.at[pl.ds(off, nel)], targ_vm, sem)

    def wait_in(pred_vm, targ_vm, sem):
        pltpu.make_async_copy(pred_hbm.at[pl.ds(0, nel)], pred_vm, sem).wait()
        pltpu.make_async_copy(targ_hbm.at[pl.ds(0, nel)], targ_vm, sem).wait()

    def wait_out(ovm, osem):
        pltpu.make_async_copy(ovm, out_hbm.at[pl.ds(0, nel)], osem).wait()

    bufs = ((pred_vm0, targ_vm0, sem0), (pred_vm1, targ_vm1, sem1))
    obufs = ((out_vm0, semo0), (out_vm1, semo1))
    start_in(0, *bufs[0])
    acc = jnp.zeros((L,), jnp.float32)
    for blk in range(NBLK):
        if blk + 1 < NBLK:
            start_in(blk + 1, *bufs[(blk + 1) % 2])
        pred_vm, targ_vm, sem = bufs[blk % 2]
        ovm, osem = obufs[blk % 2]
        if blk >= 2:
            wait_out(ovm, osem)
        wait_in(pred_vm, targ_vm, sem)
        acc = lax.fori_loop(0, NCHUNK, chunk(pred_vm, targ_vm, ovm), acc)
        off = (base_row + blk * RBLK) * NUM_CLASSES
        pltpu.async_copy(ovm, out_hbm.at[pl.ds(off, nel)], osem)
    wait_out(*obufs[0])
    wait_out(*obufs[1])
    acc_vm[...] = acc
    pltpu.sync_copy(acc_vm, part_hbm.at[wid])


def kernel(pred, target, _):
    out_flat, parts = _build_sc_kernel()(pred.reshape(-1), target.reshape(-1))
    loss = -(parts.sum() / BATCH)
    return (loss, out_flat.reshape(BATCH, NUM_CLASSES))
